# baseline (speedup 1.0000x reference)
import dataclasses
import functools

import jax
import jax.numpy as jnp
from jax import lax
from jax.experimental import pallas as pl
from jax.experimental.pallas import tpu as pltpu
from jax.experimental.pallas import tpu_sc as plsc

N = 10000
E = 320000
D = 128
NP = 10240

NC = 2
NS = 16
L = 16
NW = NC * NS
EPW = E // NW
CH = 80
NCHUNK = EPW // CH
RPT = NP // NS

_MESH = dict(core_axis_name="c", subcore_axis_name="s")

_SC_CP = pltpu.CompilerParams(
    needs_layout_passes=False, use_tc_tiling_on_sc=False)


@functools.partial(
    pl.kernel,
    out_type=jax.ShapeDtypeStruct((NC, NP), jnp.float32),
    mesh=plsc.VectorSubcoreMesh(**_MESH),
    scratch_types=[
        pltpu.VMEM((NCHUNK, CH), jnp.int32),
        pltpu.VMEM((EPW,), jnp.float32),
        pltpu.VMEM((RPT,), jnp.float32),
        pltpu.VMEM_SHARED((NP,), jnp.float32),
        pltpu.SemaphoreType.DMA,
    ],
)
def _deg_kernel(col3_hbm, ew_hbm, out_hbm, coli, ewv, zv, acc, sem):
    c = lax.axis_index("c")
    s = lax.axis_index("s")
    wid = s * NC + c

    @pl.loop(0, RPT, step=L)
    def _(i):
        zv[pl.ds(i, L)] = jnp.zeros((L,), jnp.float32)

    pltpu.sync_copy(zv, acc.at[pl.ds(s * RPT, RPT)])
    pltpu.sync_copy(col3_hbm.at[wid], coli)
    pltpu.sync_copy(ew_hbm.at[pl.ds(wid * EPW, EPW)], ewv)
    plsc.subcore_barrier()

    @pl.loop(0, NCHUNK)
    def _(k):
        pltpu.sync_copy(ewv.at[pl.ds(k * CH, CH)], acc.at[coli.at[k]],
                        add=True)

    plsc.subcore_barrier()
    pltpu.sync_copy(acc.at[pl.ds(s * RPT, RPT)],
                    out_hbm.at[c].at[pl.ds(s * RPT, RPT)])


@functools.partial(
    pl.kernel,
    out_type=jax.ShapeDtypeStruct((NC, NP, D), jnp.float32),
    mesh=plsc.VectorSubcoreMesh(**_MESH),
    scratch_types=[
        pltpu.VMEM((CH,), jnp.int32),
        pltpu.VMEM((CH,), jnp.int32),
        pltpu.VMEM((NCHUNK, CH), jnp.int32),
        pltpu.VMEM((EPW,), jnp.float32),
        pltpu.VMEM((CH, D // 2), jnp.int32),
        pltpu.VMEM((CH, D // 2), jnp.int32),
        pltpu.VMEM((CH, D), jnp.float32),
        pltpu.VMEM_SHARED((NP, D), jnp.float32),
        pltpu.SemaphoreType.DMA,
        pltpu.SemaphoreType.DMA,
        pltpu.SemaphoreType.DMA,
        pltpu.SemaphoreType.DMA,
        pltpu.SemaphoreType.DMA,
    ],
    compiler_params=_SC_CP,
)
def _agg_kernel(g_hbm, row_hbm, col3_hbm, ew_hbm, out_hbm,
                rA, rB, coli, ewv, msg0, msg1, msgf, acc,
                sem0, sem1, rsem0, rsem1, psem):
    c = lax.axis_index("c")
    s = lax.axis_index("s")
    wid = s * NC + c
    base = wid * EPW

    pltpu.async_copy(col3_hbm.at[wid], coli, psem)
    pltpu.async_copy(ew_hbm.at[pl.ds(base, EPW)], ewv, psem)

    @pl.loop(0, CH)
    def _(i):
        for j in range(D // L):
            msgf[i, pl.ds(j * L, L)] = jnp.zeros((L,), jnp.float32)

    for t in range(RPT // CH):
        pltpu.sync_copy(msgf, acc.at[pl.ds(s * RPT + t * CH, CH)])

    pltpu.make_async_copy(col3_hbm.at[wid], coli, psem).wait()
    pltpu.make_async_copy(ew_hbm.at[pl.ds(base, EPW)], ewv, psem).wait()
    plsc.subcore_barrier()

    def ridx_start(k, buf, sem):
        kc = jnp.minimum(k, NCHUNK - 1)
        pltpu.async_copy(row_hbm.at[pl.ds(base + kc * CH, CH)], buf, sem)

    def ridx_wait(buf, sem):
        pltpu.make_async_copy(row_hbm.at[pl.ds(base, CH)], buf, sem).wait()

    def gather_start(buf, idx, sem):
        pltpu.async_copy(g_hbm.at[idx], buf, sem)

    def gather_wait(buf, idx, sem):
        pltpu.make_async_copy(g_hbm.at[idx], buf, sem).wait()

    def scale(buf, k):
        @plsc.parallel_loop(0, CH, step=L)
        def _(e0):
            sv16 = ewv[pl.ds(k * CH + e0, L)]
            for u in range(L):
                sv = jnp.full((L,), sv16[u])
                for j in range(D // (2 * L)):
                    w = buf[e0 + u, pl.ds(L * j, L)]
                    v = plsc.bitcast(w, jnp.bfloat16)
                    a, b = plsc.unpack(v, format=plsc.PackFormat.INTERLEAVED)
                    msgf[e0 + u, pl.ds(2 * L * j, L)] = a * sv
                    msgf[e0 + u, pl.ds(2 * L * j + L, L)] = b * sv

    def scatter(buf, k):
        pltpu.sync_copy(msgf, acc.at[coli.at[k]], add=True)

    ridx_start(0, rA, rsem0)
    ridx_start(1, rB, rsem1)
    ridx_wait(rA, rsem0)
    gather_start(msg0, rA, sem0)
    ridx_wait(rB, rsem1)
    gather_start(msg1, rB, sem1)

    @pl.loop(0, NCHUNK - 1, step=2)
    def _(k):
        gather_wait(msg0, rA, sem0)
        ridx_start(k + 2, rA, rsem0)
        scale(msg0, k)
        scatter(msg0, k)
        ridx_wait(rA, rsem0)
        gather_start(msg0, rA, sem0)
        gather_wait(msg1, rB, sem1)
        ridx_start(k + 3, rB, rsem1)
        scale(msg1, k + 1)
        scatter(msg1, k + 1)
        ridx_wait(rB, rsem1)
        gather_start(msg1, rB, sem1)

    gather_wait(msg0, rA, sem0)
    scale(msg0, NCHUNK - 1)
    scatter(msg0, NCHUNK - 1)
    gather_wait(msg1, rB, sem1)

    plsc.subcore_barrier()
    pltpu.sync_copy(acc.at[pl.ds(s * RPT, RPT)],
                    out_hbm.at[c].at[pl.ds(s * RPT, RPT)])


_R = 1000


def _mm_first_body(x_ref, w_ref, degp_ref, g_ref, dinv_ref):
    deg = 1.0 + degp_ref[:, 0] + degp_ref[:, 1]
    dinv = lax.rsqrt(deg)
    h = jnp.dot(x_ref[...], w_ref[...], preferred_element_type=jnp.float32)
    g_ref[...] = h * dinv[:, None]
    dinv_ref[...] = dinv[:, None]


def _mm_first(x, W1, degpT):
    return pl.pallas_call(
        _mm_first_body,
        grid=(N // _R,),
        in_specs=[
            pl.BlockSpec((_R, D), lambda i: (i, 0)),
            pl.BlockSpec((D, D), lambda i: (0, 0)),
            pl.BlockSpec((_R, NC), lambda i: (i, 0)),
        ],
        out_specs=[
            pl.BlockSpec((_R, D), lambda i: (i, 0)),
            pl.BlockSpec((_R, 1), lambda i: (i, 0)),
        ],
        out_shape=[
            jax.ShapeDtypeStruct((N, D), jnp.float32),
            jax.ShapeDtypeStruct((N, 1), jnp.float32),
        ],
    )(x, W1, degpT)


def _mm_mid_body(acc_ref, g_ref, dinv_ref, b_ref, w_ref, g2_ref):
    a = acc_ref[0] + acc_ref[1] + g_ref[...]
    dinv = dinv_ref[...]
    out1 = a * dinv + b_ref[0, :][None, :]
    x2 = jnp.maximum(out1, 0.0)
    h2 = jnp.dot(x2, w_ref[...], preferred_element_type=jnp.float32)
    g2_ref[...] = h2 * dinv


def _mm_mid(acc1, g1, dinv, b1, W2):
    return pl.pallas_call(
        _mm_mid_body,
        grid=(N // _R,),
        in_specs=[
            pl.BlockSpec((NC, _R, D), lambda i: (0, i, 0)),
            pl.BlockSpec((_R, D), lambda i: (i, 0)),
            pl.BlockSpec((_R, 1), lambda i: (i, 0)),
            pl.BlockSpec((1, D), lambda i: (0, 0)),
            pl.BlockSpec((D, D), lambda i: (0, 0)),
        ],
        out_specs=pl.BlockSpec((_R, D), lambda i: (i, 0)),
        out_shape=jax.ShapeDtypeStruct((N, D), jnp.float32),
    )(acc1, g1, dinv, b1, W2)


def _mm_last_body(acc_ref, g_ref, dinv_ref, b_ref, out_ref):
    a = acc_ref[0] + acc_ref[1] + g_ref[...]
    out_ref[...] = a * dinv_ref[...] + b_ref[0, :][None, :]


def _mm_last(acc2, g2, dinv, b2):
    return pl.pallas_call(
        _mm_last_body,
        grid=(N // _R,),
        in_specs=[
            pl.BlockSpec((NC, _R, D), lambda i: (0, i, 0)),
            pl.BlockSpec((_R, D), lambda i: (i, 0)),
            pl.BlockSpec((_R, 1), lambda i: (i, 0)),
            pl.BlockSpec((1, D), lambda i: (0, 0)),
        ],
        out_specs=pl.BlockSpec((_R, D), lambda i: (i, 0)),
        out_shape=jax.ShapeDtypeStruct((N, D), jnp.float32),
    )(acc2, g2, dinv, b2)


@jax.jit
def kernel(x, edge_index, edge_weight, W1, b1, W2, b2):
    row = edge_index[0].astype(jnp.int32)
    col3 = edge_index[1].astype(jnp.int32).reshape(NW, NCHUNK, CH)
    ew = edge_weight.astype(jnp.float32)

    def to_bf16_interleaved(g):
        gb = (g.reshape(N, D // 32, 2, 16).transpose(0, 1, 3, 2)
              .reshape(N, D // 2, 2).astype(jnp.bfloat16))
        return lax.bitcast_convert_type(gb, jnp.int32)

    degp = _deg_kernel(col3, ew)
    g1, dinv = _mm_first(x, W1, degp.T)
    acc1 = _agg_kernel(to_bf16_interleaved(g1), row, col3, ew)
    g2 = _mm_mid(acc1, g1, dinv, b1.reshape(1, D), W2)
    acc2 = _agg_kernel(to_bf16_interleaved(g2), row, col3, ew)
    return _mm_last(acc2, g2, dinv, b2.reshape(1, D))


# speedup vs baseline: 1.7671x; 1.7671x over previous
import dataclasses
import functools

import jax
import jax.numpy as jnp
from jax import lax
from jax.experimental import pallas as pl
from jax.experimental.pallas import tpu as pltpu
from jax.experimental.pallas import tpu_sc as plsc

N = 10000
E = 320000
D = 128
NP = 10240

NC = 2
NS = 16
L = 16
NW = NC * NS
EPW = E // NW
CH = 80
NCHUNK = EPW // CH
RPT = NP // NS

_MESH = dict(core_axis_name="c", subcore_axis_name="s")


@functools.partial(
    pl.kernel,
    out_type=jax.ShapeDtypeStruct((NC, NP), jnp.float32),
    mesh=plsc.VectorSubcoreMesh(**_MESH),
    scratch_types=[
        pltpu.VMEM((NCHUNK, CH), jnp.int32),
        pltpu.VMEM((EPW,), jnp.float32),
        pltpu.VMEM((RPT,), jnp.float32),
        pltpu.VMEM_SHARED((NP,), jnp.float32),
        pltpu.SemaphoreType.DMA,
    ],
)
def _deg_kernel(col3_hbm, ew_hbm, out_hbm, coli, ewv, zv, acc, sem):
    c = lax.axis_index("c")
    s = lax.axis_index("s")
    wid = s * NC + c

    @pl.loop(0, RPT, step=L)
    def _(i):
        zv[pl.ds(i, L)] = jnp.zeros((L,), jnp.float32)

    pltpu.sync_copy(zv, acc.at[pl.ds(s * RPT, RPT)])
    pltpu.sync_copy(col3_hbm.at[wid], coli)
    pltpu.sync_copy(ew_hbm.at[pl.ds(wid * EPW, EPW)], ewv)
    plsc.subcore_barrier()

    @pl.loop(0, NCHUNK)
    def _(k):
        pltpu.sync_copy(ewv.at[pl.ds(k * CH, CH)], acc.at[coli.at[k]],
                        add=True)

    plsc.subcore_barrier()
    pltpu.sync_copy(acc.at[pl.ds(s * RPT, RPT)],
                    out_hbm.at[c].at[pl.ds(s * RPT, RPT)])


@functools.partial(
    pl.kernel,
    out_type=jax.ShapeDtypeStruct((NC, NP, D), jnp.float32),
    mesh=plsc.VectorSubcoreMesh(**_MESH),
    scratch_types=[
        pltpu.VMEM((CH,), jnp.int32),
        pltpu.VMEM((CH,), jnp.int32),
        pltpu.VMEM((NCHUNK, CH), jnp.int32),
        pltpu.VMEM((EPW,), jnp.float32),
        pltpu.VMEM((CH, D), jnp.float32),
        pltpu.VMEM((CH, D), jnp.float32),
        pltpu.VMEM_SHARED((NP, D), jnp.float32),
        pltpu.SemaphoreType.DMA,
        pltpu.SemaphoreType.DMA,
        pltpu.SemaphoreType.DMA,
        pltpu.SemaphoreType.DMA,
        pltpu.SemaphoreType.DMA,
    ],
)
def _agg_kernel(g_hbm, row_hbm, col3_hbm, ew_hbm, out_hbm,
                rA, rB, coli, ewv, msg0, msg1, acc,
                sem0, sem1, rsem0, rsem1, psem):
    c = lax.axis_index("c")
    s = lax.axis_index("s")
    wid = s * NC + c
    base = wid * EPW

    pltpu.async_copy(col3_hbm.at[wid], coli, psem)
    pltpu.async_copy(ew_hbm.at[pl.ds(base, EPW)], ewv, psem)

    @pl.loop(0, CH)
    def _(i):
        for j in range(D // L):
            msg0[i, pl.ds(j * L, L)] = jnp.zeros((L,), jnp.float32)

    for t in range(RPT // CH):
        pltpu.sync_copy(msg0, acc.at[pl.ds(s * RPT + t * CH, CH)])

    pltpu.make_async_copy(col3_hbm.at[wid], coli, psem).wait()
    pltpu.make_async_copy(ew_hbm.at[pl.ds(base, EPW)], ewv, psem).wait()
    plsc.subcore_barrier()

    def ridx_start(k, buf, sem):
        kc = jnp.minimum(k, NCHUNK - 1)
        pltpu.async_copy(row_hbm.at[pl.ds(base + kc * CH, CH)], buf, sem)

    def ridx_wait(buf, sem):
        pltpu.make_async_copy(row_hbm.at[pl.ds(base, CH)], buf, sem).wait()

    def gather_start(buf, idx, sem):
        pltpu.async_copy(g_hbm.at[idx], buf, sem)

    def gather_wait(buf, idx, sem):
        pltpu.make_async_copy(g_hbm.at[idx], buf, sem).wait()

    def scale(buf, k):
        @plsc.parallel_loop(0, CH, step=L)
        def _(e0):
            sv16 = ewv[pl.ds(k * CH + e0, L)]
            for u in range(L):
                sv = jnp.full((L,), sv16[u])
                for j in range(D // L):
                    buf[e0 + u, pl.ds(j * L, L)] = (
                        buf[e0 + u, pl.ds(j * L, L)] * sv)

    def scatter(buf, k):
        pltpu.sync_copy(buf, acc.at[coli.at[k]], add=True)

    ridx_start(0, rA, rsem0)
    ridx_start(1, rB, rsem1)
    ridx_wait(rA, rsem0)
    gather_start(msg0, rA, sem0)
    ridx_wait(rB, rsem1)
    gather_start(msg1, rB, sem1)

    @pl.loop(0, NCHUNK - 1, step=2)
    def _(k):
        gather_wait(msg0, rA, sem0)
        ridx_start(k + 2, rA, rsem0)
        scale(msg0, k)
        scatter(msg0, k)
        ridx_wait(rA, rsem0)
        gather_start(msg0, rA, sem0)
        gather_wait(msg1, rB, sem1)
        ridx_start(k + 3, rB, rsem1)
        scale(msg1, k + 1)
        scatter(msg1, k + 1)
        ridx_wait(rB, rsem1)
        gather_start(msg1, rB, sem1)

    gather_wait(msg0, rA, sem0)
    scale(msg0, NCHUNK - 1)
    scatter(msg0, NCHUNK - 1)
    gather_wait(msg1, rB, sem1)

    plsc.subcore_barrier()
    pltpu.sync_copy(acc.at[pl.ds(s * RPT, RPT)],
                    out_hbm.at[c].at[pl.ds(s * RPT, RPT)])


_R = 1000


def _mm_first_body(x_ref, w_ref, degp_ref, g_ref, dinv_ref):
    deg = 1.0 + degp_ref[:, 0] + degp_ref[:, 1]
    dinv = lax.rsqrt(deg)
    h = jnp.dot(x_ref[...], w_ref[...], preferred_element_type=jnp.float32)
    g_ref[...] = h * dinv[:, None]
    dinv_ref[...] = dinv[:, None]


def _mm_first(x, W1, degpT):
    return pl.pallas_call(
        _mm_first_body,
        grid=(N // _R,),
        in_specs=[
            pl.BlockSpec((_R, D), lambda i: (i, 0)),
            pl.BlockSpec((D, D), lambda i: (0, 0)),
            pl.BlockSpec((_R, NC), lambda i: (i, 0)),
        ],
        out_specs=[
            pl.BlockSpec((_R, D), lambda i: (i, 0)),
            pl.BlockSpec((_R, 1), lambda i: (i, 0)),
        ],
        out_shape=[
            jax.ShapeDtypeStruct((N, D), jnp.float32),
            jax.ShapeDtypeStruct((N, 1), jnp.float32),
        ],
    )(x, W1, degpT)


def _mm_mid_body(acc_ref, g_ref, dinv_ref, b_ref, w_ref, g2_ref):
    a = acc_ref[0] + acc_ref[1] + g_ref[...]
    dinv = dinv_ref[...]
    out1 = a * dinv + b_ref[0, :][None, :]
    x2 = jnp.maximum(out1, 0.0)
    h2 = jnp.dot(x2, w_ref[...], preferred_element_type=jnp.float32)
    g2_ref[...] = h2 * dinv


def _mm_mid(acc1, g1, dinv, b1, W2):
    return pl.pallas_call(
        _mm_mid_body,
        grid=(N // _R,),
        in_specs=[
            pl.BlockSpec((NC, _R, D), lambda i: (0, i, 0)),
            pl.BlockSpec((_R, D), lambda i: (i, 0)),
            pl.BlockSpec((_R, 1), lambda i: (i, 0)),
            pl.BlockSpec((1, D), lambda i: (0, 0)),
            pl.BlockSpec((D, D), lambda i: (0, 0)),
        ],
        out_specs=pl.BlockSpec((_R, D), lambda i: (i, 0)),
        out_shape=jax.ShapeDtypeStruct((N, D), jnp.float32),
    )(acc1, g1, dinv, b1, W2)


def _mm_last_body(acc_ref, g_ref, dinv_ref, b_ref, out_ref):
    a = acc_ref[0] + acc_ref[1] + g_ref[...]
    out_ref[...] = a * dinv_ref[...] + b_ref[0, :][None, :]


def _mm_last(acc2, g2, dinv, b2):
    return pl.pallas_call(
        _mm_last_body,
        grid=(N // _R,),
        in_specs=[
            pl.BlockSpec((NC, _R, D), lambda i: (0, i, 0)),
            pl.BlockSpec((_R, D), lambda i: (i, 0)),
            pl.BlockSpec((_R, 1), lambda i: (i, 0)),
            pl.BlockSpec((1, D), lambda i: (0, 0)),
        ],
        out_specs=pl.BlockSpec((_R, D), lambda i: (i, 0)),
        out_shape=jax.ShapeDtypeStruct((N, D), jnp.float32),
    )(acc2, g2, dinv, b2)


@jax.jit
def kernel(x, edge_index, edge_weight, W1, b1, W2, b2):
    row = edge_index[0].astype(jnp.int32)
    col3 = edge_index[1].astype(jnp.int32).reshape(NW, NCHUNK, CH)
    ew = edge_weight.astype(jnp.float32)

    degp = _deg_kernel(col3, ew)
    g1, dinv = _mm_first(x, W1, degp.T)
    acc1 = _agg_kernel(g1, row, col3, ew)
    g2 = _mm_mid(acc1, g1, dinv, b1.reshape(1, D), W2)
    acc2 = _agg_kernel(g2, row, col3, ew)
    return _mm_last(acc2, g2, dinv, b2.reshape(1, D))


# speedup vs baseline: 1.9673x; 1.1133x over previous
import dataclasses
import functools

import jax
import jax.numpy as jnp
from jax import lax
from jax.experimental import pallas as pl
from jax.experimental.pallas import tpu as pltpu
from jax.experimental.pallas import tpu_sc as plsc

N = 10000
E = 320000
D = 128
NP = 10240

NC = 2
NS = 16
L = 16
NW = NC * NS
EPW = E // NW
CH = 80
NCHUNK = EPW // CH
RPT = NP // NS

_MESH = dict(core_axis_name="c", subcore_axis_name="s")


@functools.partial(
    pl.kernel,
    out_type=jax.ShapeDtypeStruct((NC, NP), jnp.float32),
    mesh=plsc.VectorSubcoreMesh(**_MESH),
    scratch_types=[
        pltpu.VMEM((NCHUNK, CH), jnp.int32),
        pltpu.VMEM((EPW,), jnp.float32),
        pltpu.VMEM((RPT,), jnp.float32),
        pltpu.VMEM_SHARED((NP,), jnp.float32),
        pltpu.SemaphoreType.DMA,
    ],
)
def _deg_kernel(col3_hbm, ew_hbm, out_hbm, coli, ewv, zv, acc, sem):
    c = lax.axis_index("c")
    s = lax.axis_index("s")
    wid = s * NC + c

    @pl.loop(0, RPT, step=L)
    def _(i):
        zv[pl.ds(i, L)] = jnp.zeros((L,), jnp.float32)

    pltpu.sync_copy(zv, acc.at[pl.ds(s * RPT, RPT)])
    pltpu.sync_copy(col3_hbm.at[wid], coli)
    pltpu.sync_copy(ew_hbm.at[pl.ds(wid * EPW, EPW)], ewv)
    plsc.subcore_barrier()

    @pl.loop(0, NCHUNK)
    def _(k):
        pltpu.sync_copy(ewv.at[pl.ds(k * CH, CH)], acc.at[coli.at[k]],
                        add=True)

    plsc.subcore_barrier()
    pltpu.sync_copy(acc.at[pl.ds(s * RPT, RPT)],
                    out_hbm.at[c].at[pl.ds(s * RPT, RPT)])


@functools.partial(
    pl.kernel,
    out_type=jax.ShapeDtypeStruct((NC, NP, D), jnp.float32),
    mesh=plsc.VectorSubcoreMesh(**_MESH),
    scratch_types=[
        [pltpu.VMEM((CH,), jnp.int32)] * 3,
        [pltpu.VMEM((CH,), jnp.int32)] * 3,
        pltpu.VMEM((EPW,), jnp.float32),
        [pltpu.VMEM((CH, D), jnp.float32)] * 3,
        pltpu.VMEM_SHARED((NP, D), jnp.float32),
        [pltpu.SemaphoreType.DMA] * 3,
        [pltpu.SemaphoreType.DMA] * 3,
        pltpu.SemaphoreType.DMA,
    ],
)
def _agg_kernel(g_hbm, row_hbm, col3_hbm, ew_hbm, out_hbm,
                rbuf, cbuf, ewv, msg, acc, gsem, isem, psem):
    c = lax.axis_index("c")
    s = lax.axis_index("s")
    wid = s * NC + c
    base = wid * EPW

    pltpu.async_copy(ew_hbm.at[pl.ds(base, EPW)], ewv, psem)

    @pl.loop(0, CH)
    def _(i):
        for j in range(D // L):
            msg[0][i, pl.ds(j * L, L)] = jnp.zeros((L,), jnp.float32)

    for t in range(RPT // CH):
        pltpu.sync_copy(msg[0], acc.at[pl.ds(s * RPT + t * CH, CH)])

    pltpu.make_async_copy(ew_hbm.at[pl.ds(base, EPW)], ewv, psem).wait()
    plsc.subcore_barrier()

    def idx_start(k, u):
        kc = jnp.minimum(k, NCHUNK - 1)
        pltpu.async_copy(row_hbm.at[pl.ds(base + kc * CH, CH)], rbuf[u],
                         isem[u])
        pltpu.async_copy(col3_hbm.at[wid].at[kc], cbuf[u], isem[u])

    def idx_wait(u):
        pltpu.make_async_copy(row_hbm.at[pl.ds(base, CH)], rbuf[u],
                              isem[u]).wait()
        pltpu.make_async_copy(row_hbm.at[pl.ds(base, CH)], cbuf[u],
                              isem[u]).wait()

    def gather_start(u):
        pltpu.async_copy(g_hbm.at[rbuf[u]], msg[u], gsem[u])

    def gather_wait(u):
        pltpu.make_async_copy(g_hbm.at[rbuf[u]], msg[u], gsem[u]).wait()

    def scale(u, k):
        @plsc.parallel_loop(0, CH, step=L)
        def _(e0):
            sv16 = ewv[pl.ds(k * CH + e0, L)]
            for v in range(L):
                sv = jnp.full((L,), sv16[v])
                for j in range(D // L):
                    msg[u][e0 + v, pl.ds(j * L, L)] = (
                        msg[u][e0 + v, pl.ds(j * L, L)] * sv)

    def scatter(u):
        pltpu.sync_copy(msg[u], acc.at[cbuf[u]], add=True)

    idx_start(0, 0)
    idx_start(1, 1)
    idx_start(2, 2)
    idx_wait(0)
    gather_start(0)
    idx_wait(1)
    gather_start(1)

    @pl.loop(0, NCHUNK - 2, step=3)
    def _(k):
        for u in range(3):
            kk = k + u
            idx_wait((u + 2) % 3)
            gather_start((u + 2) % 3)
            gather_wait(u)
            scale(u, kk)
            scatter(u)
            idx_start(kk + 3, u)

    for kk in (NCHUNK - 2, NCHUNK - 1):
        u = kk % 3
        gather_wait(u)
        scale(u, kk)
        scatter(u)
    idx_wait((NCHUNK - 3) % 3)

    plsc.subcore_barrier()
    pltpu.sync_copy(acc.at[pl.ds(s * RPT, RPT)],
                    out_hbm.at[c].at[pl.ds(s * RPT, RPT)])


_R = 1000


def _mm_first_body(x_ref, w_ref, degp_ref, g_ref, dinv_ref):
    deg = 1.0 + degp_ref[:, 0] + degp_ref[:, 1]
    dinv = lax.rsqrt(deg)
    h = jnp.dot(x_ref[...], w_ref[...], preferred_element_type=jnp.float32)
    g_ref[...] = h * dinv[:, None]
    dinv_ref[...] = dinv[:, None]


def _mm_first(x, W1, degpT):
    return pl.pallas_call(
        _mm_first_body,
        grid=(N // _R,),
        in_specs=[
            pl.BlockSpec((_R, D), lambda i: (i, 0)),
            pl.BlockSpec((D, D), lambda i: (0, 0)),
            pl.BlockSpec((_R, NC), lambda i: (i, 0)),
        ],
        out_specs=[
            pl.BlockSpec((_R, D), lambda i: (i, 0)),
            pl.BlockSpec((_R, 1), lambda i: (i, 0)),
        ],
        out_shape=[
            jax.ShapeDtypeStruct((N, D), jnp.float32),
            jax.ShapeDtypeStruct((N, 1), jnp.float32),
        ],
    )(x, W1, degpT)


def _mm_mid_body(acc_ref, g_ref, dinv_ref, b_ref, w_ref, g2_ref):
    a = acc_ref[0] + acc_ref[1] + g_ref[...]
    dinv = dinv_ref[...]
    out1 = a * dinv + b_ref[0, :][None, :]
    x2 = jnp.maximum(out1, 0.0)
    h2 = jnp.dot(x2, w_ref[...], preferred_element_type=jnp.float32)
    g2_ref[...] = h2 * dinv


def _mm_mid(acc1, g1, dinv, b1, W2):
    return pl.pallas_call(
        _mm_mid_body,
        grid=(N // _R,),
        in_specs=[
            pl.BlockSpec((NC, _R, D), lambda i: (0, i, 0)),
            pl.BlockSpec((_R, D), lambda i: (i, 0)),
            pl.BlockSpec((_R, 1), lambda i: (i, 0)),
            pl.BlockSpec((1, D), lambda i: (0, 0)),
            pl.BlockSpec((D, D), lambda i: (0, 0)),
        ],
        out_specs=pl.BlockSpec((_R, D), lambda i: (i, 0)),
        out_shape=jax.ShapeDtypeStruct((N, D), jnp.float32),
    )(acc1, g1, dinv, b1, W2)


def _mm_last_body(acc_ref, g_ref, dinv_ref, b_ref, out_ref):
    a = acc_ref[0] + acc_ref[1] + g_ref[...]
    out_ref[...] = a * dinv_ref[...] + b_ref[0, :][None, :]


def _mm_last(acc2, g2, dinv, b2):
    return pl.pallas_call(
        _mm_last_body,
        grid=(N // _R,),
        in_specs=[
            pl.BlockSpec((NC, _R, D), lambda i: (0, i, 0)),
            pl.BlockSpec((_R, D), lambda i: (i, 0)),
            pl.BlockSpec((_R, 1), lambda i: (i, 0)),
            pl.BlockSpec((1, D), lambda i: (0, 0)),
        ],
        out_specs=pl.BlockSpec((_R, D), lambda i: (i, 0)),
        out_shape=jax.ShapeDtypeStruct((N, D), jnp.float32),
    )(acc2, g2, dinv, b2)


@jax.jit
def kernel(x, edge_index, edge_weight, W1, b1, W2, b2):
    row = edge_index[0].astype(jnp.int32)
    col3 = edge_index[1].astype(jnp.int32).reshape(NW, NCHUNK, CH)
    ew = edge_weight.astype(jnp.float32)

    degp = _deg_kernel(col3, ew)
    g1, dinv = _mm_first(x, W1, degp.T)
    acc1 = _agg_kernel(g1, row, col3, ew)
    g2 = _mm_mid(acc1, g1, dinv, b1.reshape(1, D), W2)
    acc2 = _agg_kernel(g2, row, col3, ew)
    return _mm_last(acc2, g2, dinv, b2.reshape(1, D))
